# baseline (speedup 1.0000x reference)
import functools

import jax
import jax.numpy as jnp
from jax import lax
from jax.experimental import pallas as pl
from jax.experimental.pallas import tpu as pltpu
from jax.experimental.pallas import tpu_sc as plsc

_NC = 2
_NS = 16
_NW = _NC * _NS

_BLK = 128
_CH_G = 24
_CH_S = 32


def _round_up(a, m):
    return (a + m - 1) // m * m


def _sc_row_gather(src, idx3, n_rows, d, ch, n_chunks):
    mesh = plsc.VectorSubcoreMesh(core_axis_name="c", subcore_axis_name="s")

    @functools.partial(
        pl.kernel,
        out_type=jax.ShapeDtypeStruct((n_rows, d), jnp.float32),
        mesh=mesh,
        scratch_types=[
            pltpu.VMEM((n_chunks, ch), jnp.int32),
            pltpu.VMEM((ch, d), jnp.float32),
            pltpu.VMEM((ch, d), jnp.float32),
            pltpu.SemaphoreType.DMA,
            pltpu.SemaphoreType.DMA,
        ],
    )
    def gk(idx_hbm, src_hbm, out_hbm, idx_v, rows0, rows1, sem0, sem1):
        wid = lax.axis_index("s") * _NC + lax.axis_index("c")
        base = wid * (n_chunks * ch)
        pltpu.sync_copy(idx_hbm.at[wid], idx_v)

        def g(j, rows, sem):
            return pltpu.make_async_copy(src_hbm.at[idx_v.at[j]], rows, sem)

        g(0, rows0, sem0).start()
        g(1, rows1, sem1).start()

        def body(h, carry):
            j = h * 2
            g(j, rows0, sem0).wait()
            pltpu.sync_copy(rows0, out_hbm.at[pl.ds(base + j * ch, ch)])

            @pl.when(j + 2 < n_chunks)
            def _():
                g(j + 2, rows0, sem0).start()

            g(j + 1, rows1, sem1).wait()
            pltpu.sync_copy(rows1, out_hbm.at[pl.ds(base + (j + 1) * ch, ch)])

            @pl.when(j + 3 < n_chunks)
            def _():
                g(j + 3, rows1, sem1).start()

            return carry

        lax.fori_loop(0, n_chunks // 2, body, 0)

    return gk(idx3, src)


def _mm_body(eid_ref, nt_ref, xg_ref, w_ref, b_ref, z_ref):
    t = pl.program_id(0)

    @pl.when(t < nt_ref[0])
    def _():
        acc = lax.dot_general(
            xg_ref[...], w_ref[0],
            (((1,), (1,)), ((), ())),
            preferred_element_type=jnp.float32,
        )
        z_ref[...] = acc + b_ref[0]


def _grouped_matmul(x_g, w_enc, b_enc, tile_e, ntiles, b_pad, d_in, d_lat):
    t_grid = b_pad // _BLK
    grid_spec = pltpu.PrefetchScalarGridSpec(
        num_scalar_prefetch=2,
        grid=(t_grid,),
        in_specs=[
            pl.BlockSpec((_BLK, d_in), lambda t, e, n: (t, 0)),
            pl.BlockSpec((1, d_lat, d_in), lambda t, e, n: (e[t], 0, 0)),
            pl.BlockSpec((1, 1, d_lat), lambda t, e, n: (e[t], 0, 0)),
        ],
        out_specs=pl.BlockSpec((_BLK, d_lat), lambda t, e, n: (t, 0)),
    )
    return pl.pallas_call(
        _mm_body,
        grid_spec=grid_spec,
        out_shape=jax.ShapeDtypeStruct((b_pad, d_lat), jnp.float32),
    )(tile_e, ntiles, x_g, w_enc, b_enc.reshape(b_enc.shape[0], 1, d_lat))


def _dec_body(z_ref, w_ref, b_ref, y_ref):
    y_ref[...] = (
        jnp.dot(z_ref[...], w_ref[...], preferred_element_type=jnp.float32)
        + b_ref[0, 0]
    )


def _decoder(z, w_dec_col, b_dec, b, d_lat):
    rows = 1024
    return pl.pallas_call(
        _dec_body,
        grid=(b // rows,),
        in_specs=[
            pl.BlockSpec((rows, d_lat), lambda i: (i, 0)),
            pl.BlockSpec((d_lat, 1), lambda i: (0, 0)),
            pl.BlockSpec((1, 1), lambda i: (0, 0)),
        ],
        out_specs=pl.BlockSpec((rows, 1), lambda i: (i, 0)),
        out_shape=jax.ShapeDtypeStruct((b, 1), jnp.float32),
    )(z, w_dec_col, b_dec)


def kernel(x, species_idx, W_enc, b_enc, W_dec, b_dec):
    b, d_in = x.shape
    e, d_lat, _ = W_enc.shape

    sp = species_idx.astype(jnp.int32)
    order = jnp.argsort(sp).astype(jnp.int32)
    sps = jnp.sort(sp)
    counts = jnp.bincount(sp, length=e).astype(jnp.int32)
    padded = (counts + (_BLK - 1)) // _BLK * _BLK
    ends = jnp.cumsum(padded)
    poff = ends - padded
    starts = jnp.cumsum(counts) - counts

    b_pad = _round_up(b // _BLK * _BLK + (e - 1) * _BLK, _NW * _CH_G)
    n_ch_g = b_pad // (_NW * _CH_G)
    n_ch_s = b // (_NW * _CH_S)
    t_grid = b_pad // _BLK

    dest = poff[sps] + (jnp.arange(b, dtype=jnp.int32) - starts[sps])
    pad_fill = jnp.arange(b_pad, dtype=jnp.int32) % b
    gather_idx = pad_fill.at[dest].set(order)
    token_slot = jnp.zeros(b, jnp.int32).at[order].set(dest)

    ntiles = ((ends[e - 1] + _BLK - 1) // _BLK).astype(jnp.int32).reshape(1)
    tile_e = jnp.clip(
        jnp.searchsorted(ends, jnp.arange(t_grid, dtype=jnp.int32) * _BLK,
                         side="right"),
        0, e - 1,
    ).astype(jnp.int32)

    gidx3 = gather_idx.reshape(_NW, n_ch_g, _CH_G)
    sidx3 = token_slot.reshape(_NW, n_ch_s, _CH_S)

    x_g = _sc_row_gather(x, gidx3, b_pad, d_in, _CH_G, n_ch_g)
    z_g = _grouped_matmul(x_g, W_enc, b_enc, tile_e, ntiles, b_pad, d_in,
                          d_lat)
    z = _sc_row_gather(z_g, sidx3, b, d_lat, _CH_S, n_ch_s)
    y = _decoder(z, W_dec.reshape(1, d_lat).T, b_dec.reshape(1, 1), b, d_lat)
    return (y, z)


# speedup vs baseline: 1.3486x; 1.3486x over previous
import functools

import jax
import jax.numpy as jnp
from jax import lax
from jax.experimental import pallas as pl
from jax.experimental.pallas import tpu as pltpu
from jax.experimental.pallas import tpu_sc as plsc

_NC = 2
_NS = 16
_NW = _NC * _NS

_BLK = 256
_CH_G = 24
_CH_S = 32


def _round_up(a, m):
    return (a + m - 1) // m * m


def _sc_row_gather(src, idx3, n_rows, d, ch, n_chunks):
    mesh = plsc.VectorSubcoreMesh(core_axis_name="c", subcore_axis_name="s")

    @functools.partial(
        pl.kernel,
        out_type=jax.ShapeDtypeStruct((n_rows, d), jnp.float32),
        mesh=mesh,
        scratch_types=[
            pltpu.VMEM((n_chunks, ch), jnp.int32),
            pltpu.VMEM((ch, d), jnp.float32),
            pltpu.VMEM((ch, d), jnp.float32),
            pltpu.SemaphoreType.DMA,
            pltpu.SemaphoreType.DMA,
        ],
    )
    def gk(idx_hbm, src_hbm, out_hbm, idx_v, rows0, rows1, sem0, sem1):
        wid = lax.axis_index("s") * _NC + lax.axis_index("c")
        base = wid * (n_chunks * ch)
        pltpu.sync_copy(idx_hbm.at[wid], idx_v)

        def g(j, rows, sem):
            return pltpu.make_async_copy(src_hbm.at[idx_v.at[j]], rows, sem)

        g(0, rows0, sem0).start()
        g(1, rows1, sem1).start()

        def body(h, carry):
            j = h * 2
            g(j, rows0, sem0).wait()
            pltpu.sync_copy(rows0, out_hbm.at[pl.ds(base + j * ch, ch)])

            @pl.when(j + 2 < n_chunks)
            def _():
                g(j + 2, rows0, sem0).start()

            g(j + 1, rows1, sem1).wait()
            pltpu.sync_copy(rows1, out_hbm.at[pl.ds(base + (j + 1) * ch, ch)])

            @pl.when(j + 3 < n_chunks)
            def _():
                g(j + 3, rows1, sem1).start()

            return carry

        lax.fori_loop(0, n_chunks // 2, body, 0)

    return gk(idx3, src)


def _mm_body(eid_ref, nt_ref, xg_ref, w_ref, b_ref, z_ref):
    t = pl.program_id(0)

    @pl.when(t < nt_ref[0])
    def _():
        acc = lax.dot_general(
            xg_ref[...], w_ref[0],
            (((1,), (1,)), ((), ())),
            preferred_element_type=jnp.float32,
        )
        z_ref[...] = acc + b_ref[0]


def _grouped_matmul(x_g, w_enc, b_enc, tile_e, ntiles, b_pad, d_in, d_lat):
    t_grid = b_pad // _BLK
    grid_spec = pltpu.PrefetchScalarGridSpec(
        num_scalar_prefetch=2,
        grid=(t_grid,),
        in_specs=[
            pl.BlockSpec((_BLK, d_in), lambda t, e, n: (t, 0)),
            pl.BlockSpec((1, d_lat, d_in), lambda t, e, n: (e[t], 0, 0)),
            pl.BlockSpec((1, 1, d_lat), lambda t, e, n: (e[t], 0, 0)),
        ],
        out_specs=pl.BlockSpec((_BLK, d_lat), lambda t, e, n: (t, 0)),
    )
    return pl.pallas_call(
        _mm_body,
        grid_spec=grid_spec,
        out_shape=jax.ShapeDtypeStruct((b_pad, d_lat), jnp.float32),
    )(tile_e, ntiles, x_g, w_enc, b_enc.reshape(b_enc.shape[0], 1, d_lat))


def _dec_body(z_ref, w_ref, b_ref, y_ref):
    y_ref[...] = (
        jnp.dot(z_ref[...], w_ref[...], preferred_element_type=jnp.float32)
        + b_ref[0, 0]
    )


def _decoder(z, w_dec_col, b_dec, b, d_lat):
    rows = 1024
    return pl.pallas_call(
        _dec_body,
        grid=(b // rows,),
        in_specs=[
            pl.BlockSpec((rows, d_lat), lambda i: (i, 0)),
            pl.BlockSpec((d_lat, 1), lambda i: (0, 0)),
            pl.BlockSpec((1, 1), lambda i: (0, 0)),
        ],
        out_specs=pl.BlockSpec((rows, 1), lambda i: (i, 0)),
        out_shape=jax.ShapeDtypeStruct((b, 1), jnp.float32),
    )(z, w_dec_col, b_dec)


def kernel(x, species_idx, W_enc, b_enc, W_dec, b_dec):
    b, d_in = x.shape
    e, d_lat, _ = W_enc.shape

    sp = species_idx.astype(jnp.int32)
    onehot = (sp[:, None] == jnp.arange(e, dtype=jnp.int32)[None, :])
    pos = jnp.cumsum(onehot.astype(jnp.int32), axis=0)
    counts = pos[-1]
    rank = jnp.take_along_axis(pos, sp[:, None], axis=1)[:, 0] - 1
    padded = (counts + (_BLK - 1)) // _BLK * _BLK
    ends = jnp.cumsum(padded)
    poff = ends - padded

    b_pad = _round_up(b // _BLK * _BLK + (e - 1) * _BLK, _NW * _CH_G)
    n_ch_g = b_pad // (_NW * _CH_G)
    n_ch_s = b // (_NW * _CH_S)
    t_grid = b_pad // _BLK

    token_slot = poff[sp] + rank
    pad_fill = jnp.arange(b_pad, dtype=jnp.int32) % b
    gather_idx = pad_fill.at[token_slot].set(
        jnp.arange(b, dtype=jnp.int32))

    ntiles = ((ends[e - 1] + _BLK - 1) // _BLK).astype(jnp.int32).reshape(1)
    tile_e = jnp.clip(
        jnp.searchsorted(ends, jnp.arange(t_grid, dtype=jnp.int32) * _BLK,
                         side="right"),
        0, e - 1,
    ).astype(jnp.int32)

    gidx3 = gather_idx.reshape(_NW, n_ch_g, _CH_G)
    sidx3 = token_slot.reshape(_NW, n_ch_s, _CH_S)

    x_g = _sc_row_gather(x, gidx3, b_pad, d_in, _CH_G, n_ch_g)
    z_g = _grouped_matmul(x_g, W_enc, b_enc, tile_e, ntiles, b_pad, d_in,
                          d_lat)
    z = _sc_row_gather(z_g, sidx3, b, d_lat, _CH_S, n_ch_s)
    y = _decoder(z, W_dec.reshape(1, d_lat).T, b_dec.reshape(1, 1), b, d_lat)
    return (y, z)
